# Initial kernel scaffold; baseline (speedup 1.0000x reference)
import functools

import jax
import jax.numpy as jnp
from jax import lax
from jax.experimental import pallas as pl
from jax.experimental.pallas import tpu as pltpu
from jax.experimental.pallas import tpu_sc as plsc

N = 10000
E = 320000
F = 128
C = 64

NPAD = 10240
ROWS_PER_TILE = NPAD // 16
CHUNK = 128
NCHK = ((E + CHUNK - 1) // CHUNK + 31) // 32 * 32
EPAD = NCHK * CHUNK
CHUNKS_PER_TILE = NCHK // 32
DEGW = 16


def _sc_agg_body(compute_deg, h_hbm, src_hbm, dst_hbm, *refs):
    if compute_deg:
        (agg_out, deg_out, src_v, dst_v, rows_v, ones_v, zbuf_v,
         acc_sp, deg_sp, sem) = refs
    else:
        (agg_out, src_v, dst_v, rows_v, zbuf_v, acc_sp, sem) = refs

    cid = lax.axis_index("c")
    tid = lax.axis_index("s")
    wid = cid * 16 + tid

    def fill(i, _):
        for g in range(F // 16):
            rows_v[i, pl.ds(g * 16, 16)] = jnp.zeros((16,), jnp.float32)
        zbuf_v[i, :] = jnp.zeros((DEGW,), jnp.float32)
        if compute_deg:
            ones_v[i, :] = jnp.ones((DEGW,), jnp.float32)
        return 0
    lax.fori_loop(0, CHUNK, fill, 0)

    my0 = tid * ROWS_PER_TILE
    for k in range(ROWS_PER_TILE // CHUNK):
        pltpu.sync_copy(rows_v, acc_sp.at[pl.ds(my0 + k * CHUNK, CHUNK)])
        if compute_deg:
            pltpu.sync_copy(zbuf_v, deg_sp.at[pl.ds(my0 + k * CHUNK, CHUNK)])
    plsc.subcore_barrier()

    pltpu.sync_copy(src_hbm.at[pl.ds(wid * CHUNKS_PER_TILE, CHUNKS_PER_TILE)], src_v)
    pltpu.sync_copy(dst_hbm.at[pl.ds(wid * CHUNKS_PER_TILE, CHUNKS_PER_TILE)], dst_v)

    def edge_body(j, _):
        pltpu.async_copy(h_hbm.at[src_v.at[j]], rows_v, sem).wait()
        pltpu.sync_copy(rows_v, acc_sp.at[dst_v.at[j]], add=True)
        if compute_deg:
            pltpu.sync_copy(ones_v, deg_sp.at[dst_v.at[j]], add=True)
        return 0
    lax.fori_loop(0, CHUNKS_PER_TILE, edge_body, 0)

    plsc.subcore_barrier()

    out0 = cid * NPAD + my0
    pltpu.sync_copy(acc_sp.at[pl.ds(my0, ROWS_PER_TILE)],
                    agg_out.at[pl.ds(out0, ROWS_PER_TILE)])
    if compute_deg:
        pltpu.sync_copy(deg_sp.at[pl.ds(my0, ROWS_PER_TILE)],
                        deg_out.at[pl.ds(out0, ROWS_PER_TILE)])


def _make_sc_agg(compute_deg):
    out_type = [jax.ShapeDtypeStruct((2 * NPAD, F), jnp.float32)]
    scratch = [
        pltpu.VMEM((CHUNKS_PER_TILE, CHUNK), jnp.int32),
        pltpu.VMEM((CHUNKS_PER_TILE, CHUNK), jnp.int32),
        pltpu.VMEM((CHUNK, F), jnp.float32),
    ]
    if compute_deg:
        out_type = out_type + [jax.ShapeDtypeStruct((2 * NPAD, DEGW), jnp.float32)]
        scratch = scratch + [pltpu.VMEM((CHUNK, DEGW), jnp.float32)]
    scratch = scratch + [pltpu.VMEM((CHUNK, DEGW), jnp.float32)]
    scratch = scratch + [pltpu.VMEM_SHARED((NPAD, F), jnp.float32)]
    if compute_deg:
        scratch = scratch + [pltpu.VMEM_SHARED((NPAD, DEGW), jnp.float32)]
    scratch = scratch + [pltpu.SemaphoreType.DMA]

    return pl.kernel(
        functools.partial(_sc_agg_body, compute_deg),
        out_type=out_type if compute_deg else out_type[0],
        mesh=plsc.VectorSubcoreMesh(core_axis_name="c", subcore_axis_name="s"),
        scratch_types=scratch,
    )


_sc_agg_deg = _make_sc_agg(True)
_sc_agg = _make_sc_agg(False)

_BLK = 2000


def _tc_layer1_body(x, a0, a1, d0, d1, wl, wr, b, o):
    deg = jnp.clip(d0[...] + d1[...], 1.0, None)
    mean = (a0[...] + a1[...]) / deg
    h = (jnp.dot(x[...], wl[...], preferred_element_type=jnp.float32)
         + jnp.dot(mean, wr[...], preferred_element_type=jnp.float32)
         + b[...])
    o[...] = jnp.maximum(h, 0.0)


def _tc_layer2_body(x, a0, a1, d0, d1, wl, wr, b, wm, bm, o):
    deg = jnp.clip(d0[...] + d1[...], 1.0, None)
    mean = (a0[...] + a1[...]) / deg
    h = (jnp.dot(x[...], wl[...], preferred_element_type=jnp.float32)
         + jnp.dot(mean, wr[...], preferred_element_type=jnp.float32)
         + b[...])
    h = jnp.maximum(h, 0.0)
    logits = jnp.dot(h, wm[...], preferred_element_type=jnp.float32) + bm[...]
    m = jnp.max(logits, axis=1, keepdims=True)
    s = logits - m
    lse = jnp.log(jnp.sum(jnp.exp(s), axis=1, keepdims=True))
    o[...] = s - lse


def _row_spec(w):
    return pl.BlockSpec((_BLK, w), lambda i: (i, 0))


def _full_spec(h, w):
    return pl.BlockSpec((h, w), lambda i: (0, 0))


def _tc_layer1(x, a0, a1, d0, d1, wl, wr, b):
    return pl.pallas_call(
        _tc_layer1_body,
        grid=(N // _BLK,),
        in_specs=[_row_spec(F), _row_spec(F), _row_spec(F),
                  _row_spec(1), _row_spec(1),
                  _full_spec(F, F), _full_spec(F, F), _full_spec(1, F)],
        out_specs=_row_spec(F),
        out_shape=jax.ShapeDtypeStruct((N, F), jnp.float32),
    )(x, a0, a1, d0, d1, wl, wr, b)


def _tc_layer2(x, a0, a1, d0, d1, wl, wr, b, wm, bm):
    return pl.pallas_call(
        _tc_layer2_body,
        grid=(N // _BLK,),
        in_specs=[_row_spec(F), _row_spec(F), _row_spec(F),
                  _row_spec(1), _row_spec(1),
                  _full_spec(F, F), _full_spec(F, F), _full_spec(1, F),
                  _full_spec(F, C), _full_spec(1, C)],
        out_specs=_row_spec(C),
        out_shape=jax.ShapeDtypeStruct((N, C), jnp.float32),
    )(x, a0, a1, d0, d1, wl, wr, b, wm, bm)


def kernel(x, edge_index, W1l, W1r, b1, W2l, W2r, b2, Wm, bm):
    src = edge_index[0]
    dst = edge_index[1]
    pad = EPAD - E
    src_p = jnp.concatenate([src, jnp.zeros((pad,), jnp.int32)]).reshape(NCHK, CHUNK)
    dst_p = jnp.concatenate([dst, jnp.full((pad,), N, jnp.int32)]).reshape(NCHK, CHUNK)

    agg1, deg1 = _sc_agg_deg(x, src_p, dst_p)
    a0, a1 = agg1[:N], agg1[NPAD:NPAD + N]
    d0, d1 = deg1[:N, :1], deg1[NPAD:NPAD + N, :1]

    h1 = _tc_layer1(x, a0, a1, d0, d1, W1l, W1r, b1.reshape(1, F))

    agg2 = _sc_agg(h1, src_p, dst_p)
    c0, c1 = agg2[:N], agg2[NPAD:NPAD + N]

    return _tc_layer2(h1, c0, c1, d0, d1, W2l, W2r, b2.reshape(1, F),
                      Wm, bm.reshape(1, C))


# speedup vs baseline: 2.9010x; 2.9010x over previous
import functools

import jax
import jax.numpy as jnp
from jax import lax
from jax.experimental import pallas as pl
from jax.experimental.pallas import tpu as pltpu
from jax.experimental.pallas import tpu_sc as plsc

N = 10000
E = 320000
F = 128
C = 64

NPAD = 10240
ROWS_PER_TILE = NPAD // 16
CHUNK = 128
NCHK = ((E + CHUNK - 1) // CHUNK + 255) // 256 * 256
EPAD = NCHK * CHUNK
CHUNKS_PER_TILE = NCHK // 32
DEGW = 16


def _sc_agg_body(h_hbm, src_hbm, dst_hbm, agg_out, src_v, dst_v, rows_v,
                 acc_sp, sem):
    cid = lax.axis_index("c")
    tid = lax.axis_index("s")
    wid = cid * 16 + tid

    def fill(i, _):
        for g in range(F // 16):
            rows_v[i, pl.ds(g * 16, 16)] = jnp.zeros((16,), jnp.float32)
        return 0
    lax.fori_loop(0, CHUNK, fill, 0)

    my0 = tid * ROWS_PER_TILE
    for k in range(ROWS_PER_TILE // CHUNK):
        pltpu.sync_copy(rows_v, acc_sp.at[pl.ds(my0 + k * CHUNK, CHUNK)])
    plsc.subcore_barrier()

    pltpu.sync_copy(src_hbm.at[pl.ds(wid * CHUNKS_PER_TILE, CHUNKS_PER_TILE)], src_v)
    pltpu.sync_copy(dst_hbm.at[pl.ds(wid * CHUNKS_PER_TILE, CHUNKS_PER_TILE)], dst_v)

    def edge_body(j, _):
        pltpu.async_copy(h_hbm.at[src_v.at[j]], rows_v, sem).wait()
        pltpu.sync_copy(rows_v, acc_sp.at[dst_v.at[j]], add=True)
        return 0
    lax.fori_loop(0, CHUNKS_PER_TILE, edge_body, 0)

    plsc.subcore_barrier()

    out0 = cid * NPAD + my0
    pltpu.sync_copy(acc_sp.at[pl.ds(my0, ROWS_PER_TILE)],
                    agg_out.at[pl.ds(out0, ROWS_PER_TILE)])


def _sc_deg_body(dst_hbm, deg_out, dst_v, ones_v, zbuf_v, deg_sp):
    cid = lax.axis_index("c")
    tid = lax.axis_index("s")
    wid = cid * 16 + tid

    def fill(i, _):
        ones_v[i, :] = jnp.ones((DEGW,), jnp.float32)
        zbuf_v[i, :] = jnp.zeros((DEGW,), jnp.float32)
        return 0
    lax.fori_loop(0, CHUNK, fill, 0)

    my0 = tid * ROWS_PER_TILE
    for k in range(ROWS_PER_TILE // CHUNK):
        pltpu.sync_copy(zbuf_v, deg_sp.at[pl.ds(my0 + k * CHUNK, CHUNK)])
    plsc.subcore_barrier()

    pltpu.sync_copy(dst_hbm.at[pl.ds(wid * CHUNKS_PER_TILE, CHUNKS_PER_TILE)], dst_v)

    def edge_body(j, _):
        pltpu.sync_copy(ones_v, deg_sp.at[dst_v.at[j]], add=True)
        return 0
    lax.fori_loop(0, CHUNKS_PER_TILE, edge_body, 0)

    plsc.subcore_barrier()

    out0 = cid * NPAD + my0
    pltpu.sync_copy(deg_sp.at[pl.ds(my0, ROWS_PER_TILE)],
                    deg_out.at[pl.ds(out0, ROWS_PER_TILE)])


_SC_PARAMS = pltpu.CompilerParams(use_tc_tiling_on_sc=False)
_SC_MESH = dict(core_axis_name="c", subcore_axis_name="s")

_sc_agg = pl.kernel(
    _sc_agg_body,
    out_type=jax.ShapeDtypeStruct((2 * NPAD, F), jnp.float32),
    mesh=plsc.VectorSubcoreMesh(**_SC_MESH),
    scratch_types=[
        pltpu.VMEM((CHUNKS_PER_TILE, CHUNK), jnp.int32),
        pltpu.VMEM((CHUNKS_PER_TILE, CHUNK), jnp.int32),
        pltpu.VMEM((CHUNK, F), jnp.float32),
        pltpu.VMEM_SHARED((NPAD, F), jnp.float32),
        pltpu.SemaphoreType.DMA,
    ],
    compiler_params=_SC_PARAMS,
)

_sc_deg = pl.kernel(
    _sc_deg_body,
    out_type=jax.ShapeDtypeStruct((2 * NPAD, DEGW), jnp.float32),
    mesh=plsc.VectorSubcoreMesh(**_SC_MESH),
    scratch_types=[
        pltpu.VMEM((CHUNKS_PER_TILE, CHUNK), jnp.int32),
        pltpu.VMEM((CHUNK, DEGW), jnp.float32),
        pltpu.VMEM((CHUNK, DEGW), jnp.float32),
        pltpu.VMEM_SHARED((NPAD, DEGW), jnp.float32),
    ],
    compiler_params=_SC_PARAMS,
)

_BLK = 2000


def _tc_layer1_body(x, a0, a1, d0, d1, wl, wr, b, o):
    deg = jnp.clip(d0[...] + d1[...], 1.0, None)
    mean = (a0[...] + a1[...]) / deg
    h = (jnp.dot(x[...], wl[...], preferred_element_type=jnp.float32)
         + jnp.dot(mean, wr[...], preferred_element_type=jnp.float32)
         + b[...])
    o[...] = jnp.maximum(h, 0.0)


def _tc_layer2_body(x, a0, a1, d0, d1, wl, wr, b, wm, bm, o):
    deg = jnp.clip(d0[...] + d1[...], 1.0, None)
    mean = (a0[...] + a1[...]) / deg
    h = (jnp.dot(x[...], wl[...], preferred_element_type=jnp.float32)
         + jnp.dot(mean, wr[...], preferred_element_type=jnp.float32)
         + b[...])
    h = jnp.maximum(h, 0.0)
    logits = jnp.dot(h, wm[...], preferred_element_type=jnp.float32) + bm[...]
    m = jnp.max(logits, axis=1, keepdims=True)
    s = logits - m
    lse = jnp.log(jnp.sum(jnp.exp(s), axis=1, keepdims=True))
    o[...] = s - lse


def _row_spec(w):
    return pl.BlockSpec((_BLK, w), lambda i: (i, 0))


def _full_spec(h, w):
    return pl.BlockSpec((h, w), lambda i: (0, 0))


def _tc_layer1(x, a0, a1, d0, d1, wl, wr, b):
    return pl.pallas_call(
        _tc_layer1_body,
        grid=(N // _BLK,),
        in_specs=[_row_spec(F), _row_spec(F), _row_spec(F),
                  _row_spec(1), _row_spec(1),
                  _full_spec(F, F), _full_spec(F, F), _full_spec(1, F)],
        out_specs=_row_spec(F),
        out_shape=jax.ShapeDtypeStruct((N, F), jnp.float32),
    )(x, a0, a1, d0, d1, wl, wr, b)


def _tc_layer2(x, a0, a1, d0, d1, wl, wr, b, wm, bm):
    return pl.pallas_call(
        _tc_layer2_body,
        grid=(N // _BLK,),
        in_specs=[_row_spec(F), _row_spec(F), _row_spec(F),
                  _row_spec(1), _row_spec(1),
                  _full_spec(F, F), _full_spec(F, F), _full_spec(1, F),
                  _full_spec(F, C), _full_spec(1, C)],
        out_specs=_row_spec(C),
        out_shape=jax.ShapeDtypeStruct((N, C), jnp.float32),
    )(x, a0, a1, d0, d1, wl, wr, b, wm, bm)


def kernel(x, edge_index, W1l, W1r, b1, W2l, W2r, b2, Wm, bm):
    src = edge_index[0]
    dst = edge_index[1]
    pad = EPAD - E
    src_p = jnp.concatenate([src, jnp.zeros((pad,), jnp.int32)]).reshape(NCHK, CHUNK)
    dst_p = jnp.concatenate([dst, jnp.full((pad,), N, jnp.int32)]).reshape(NCHK, CHUNK)

    agg1 = _sc_agg(x, src_p, dst_p)
    deg1 = _sc_deg(dst_p)
    a0, a1 = agg1[:N], agg1[NPAD:NPAD + N]
    d0, d1 = deg1[:N, :1], deg1[NPAD:NPAD + N, :1]

    h1 = _tc_layer1(x, a0, a1, d0, d1, W1l, W1r, b1.reshape(1, F))

    agg2 = _sc_agg(h1, src_p, dst_p)
    c0, c1 = agg2[:N], agg2[NPAD:NPAD + N]

    return _tc_layer2(h1, c0, c1, d0, d1, W2l, W2r, b2.reshape(1, F),
                      Wm, bm.reshape(1, C))


# speedup vs baseline: 3.0890x; 1.0648x over previous
import functools

import jax
import jax.numpy as jnp
from jax import lax
from jax.experimental import pallas as pl
from jax.experimental.pallas import tpu as pltpu
from jax.experimental.pallas import tpu_sc as plsc

N = 10000
E = 320000
F = 128
C = 64

NPAD = 10240
ROWS_PER_TILE = NPAD // 16
CHUNK = 128
NCHK = ((E + CHUNK - 1) // CHUNK + 255) // 256 * 256
EPAD = NCHK * CHUNK
CHUNKS_PER_TILE = NCHK // 32
IDXB = 16
DEGW = 16


def _sc_agg_body(h_hbm, src_hbm, dst_hbm, agg_out, src_v, dst_v, rows_v0,
                 rows_v1, acc_sp, sem0, sem1):
    cid = lax.axis_index("c")
    tid = lax.axis_index("s")
    wid = cid * 16 + tid

    def fill(i, _):
        for g in range(F // 16):
            rows_v0[i, pl.ds(g * 16, 16)] = jnp.zeros((16,), jnp.float32)
        return 0
    lax.fori_loop(0, CHUNK, fill, 0)

    my0 = tid * ROWS_PER_TILE
    for k in range(ROWS_PER_TILE // CHUNK):
        pltpu.sync_copy(rows_v0, acc_sp.at[pl.ds(my0 + k * CHUNK, CHUNK)])
    plsc.subcore_barrier()

    def outer(g, _):
        base = wid * CHUNKS_PER_TILE + g * IDXB
        pltpu.sync_copy(src_hbm.at[pl.ds(base, IDXB)], src_v)
        pltpu.sync_copy(dst_hbm.at[pl.ds(base, IDXB)], dst_v)
        pltpu.async_copy(h_hbm.at[src_v.at[0]], rows_v0, sem0)

        def inner(jj, _):
            j = jj * 2
            pltpu.make_async_copy(h_hbm.at[src_v.at[0]], rows_v0, sem0).wait()
            pltpu.async_copy(h_hbm.at[src_v.at[j + 1]], rows_v1, sem1)
            pltpu.sync_copy(rows_v0, acc_sp.at[dst_v.at[j]], add=True)

            pltpu.make_async_copy(h_hbm.at[src_v.at[0]], rows_v1, sem1).wait()

            @pl.when(jj + 1 < IDXB // 2)
            def _():
                pltpu.async_copy(h_hbm.at[src_v.at[j + 2]], rows_v0, sem0)
            pltpu.sync_copy(rows_v1, acc_sp.at[dst_v.at[j + 1]], add=True)
            return 0
        lax.fori_loop(0, IDXB // 2, inner, 0)
        return 0
    lax.fori_loop(0, CHUNKS_PER_TILE // IDXB, outer, 0)

    plsc.subcore_barrier()

    out0 = cid * NPAD + my0
    pltpu.sync_copy(acc_sp.at[pl.ds(my0, ROWS_PER_TILE)],
                    agg_out.at[pl.ds(out0, ROWS_PER_TILE)])


def _sc_deg_body(dst_hbm, deg_out, dst_v, ones_v, zbuf_v, deg_sp):
    cid = lax.axis_index("c")
    tid = lax.axis_index("s")
    wid = cid * 16 + tid

    def fill(i, _):
        ones_v[i, :] = jnp.ones((DEGW,), jnp.float32)
        zbuf_v[i, :] = jnp.zeros((DEGW,), jnp.float32)
        return 0
    lax.fori_loop(0, CHUNK, fill, 0)

    my0 = tid * ROWS_PER_TILE
    for k in range(ROWS_PER_TILE // CHUNK):
        pltpu.sync_copy(zbuf_v, deg_sp.at[pl.ds(my0 + k * CHUNK, CHUNK)])
    plsc.subcore_barrier()

    pltpu.sync_copy(dst_hbm.at[pl.ds(wid * CHUNKS_PER_TILE, CHUNKS_PER_TILE)], dst_v)

    def edge_body(j, _):
        pltpu.sync_copy(ones_v, deg_sp.at[dst_v.at[j]], add=True)
        return 0
    lax.fori_loop(0, CHUNKS_PER_TILE, edge_body, 0)

    plsc.subcore_barrier()

    out0 = cid * NPAD + my0
    pltpu.sync_copy(deg_sp.at[pl.ds(my0, ROWS_PER_TILE)],
                    deg_out.at[pl.ds(out0, ROWS_PER_TILE)])


_SC_PARAMS = pltpu.CompilerParams(use_tc_tiling_on_sc=False)
_SC_MESH = dict(core_axis_name="c", subcore_axis_name="s")

_sc_agg = pl.kernel(
    _sc_agg_body,
    out_type=jax.ShapeDtypeStruct((2 * NPAD, F), jnp.float32),
    mesh=plsc.VectorSubcoreMesh(**_SC_MESH),
    scratch_types=[
        pltpu.VMEM((IDXB, CHUNK), jnp.int32),
        pltpu.VMEM((IDXB, CHUNK), jnp.int32),
        pltpu.VMEM((CHUNK, F), jnp.float32),
        pltpu.VMEM((CHUNK, F), jnp.float32),
        pltpu.VMEM_SHARED((NPAD, F), jnp.float32),
        pltpu.SemaphoreType.DMA,
        pltpu.SemaphoreType.DMA,
    ],
    compiler_params=_SC_PARAMS,
)

_sc_deg = pl.kernel(
    _sc_deg_body,
    out_type=jax.ShapeDtypeStruct((2 * NPAD, DEGW), jnp.float32),
    mesh=plsc.VectorSubcoreMesh(**_SC_MESH),
    scratch_types=[
        pltpu.VMEM((CHUNKS_PER_TILE, CHUNK), jnp.int32),
        pltpu.VMEM((CHUNK, DEGW), jnp.float32),
        pltpu.VMEM((CHUNK, DEGW), jnp.float32),
        pltpu.VMEM_SHARED((NPAD, DEGW), jnp.float32),
    ],
    compiler_params=_SC_PARAMS,
)

_BLK = 2000


def _tc_layer1_body(x, a0, a1, d0, d1, wl, wr, b, o):
    deg = jnp.clip(d0[...] + d1[...], 1.0, None)
    mean = (a0[...] + a1[...]) / deg
    h = (jnp.dot(x[...], wl[...], preferred_element_type=jnp.float32)
         + jnp.dot(mean, wr[...], preferred_element_type=jnp.float32)
         + b[...])
    o[...] = jnp.maximum(h, 0.0)


def _tc_layer2_body(x, a0, a1, d0, d1, wl, wr, b, wm, bm, o):
    deg = jnp.clip(d0[...] + d1[...], 1.0, None)
    mean = (a0[...] + a1[...]) / deg
    h = (jnp.dot(x[...], wl[...], preferred_element_type=jnp.float32)
         + jnp.dot(mean, wr[...], preferred_element_type=jnp.float32)
         + b[...])
    h = jnp.maximum(h, 0.0)
    logits = jnp.dot(h, wm[...], preferred_element_type=jnp.float32) + bm[...]
    m = jnp.max(logits, axis=1, keepdims=True)
    s = logits - m
    lse = jnp.log(jnp.sum(jnp.exp(s), axis=1, keepdims=True))
    o[...] = s - lse


def _row_spec(w):
    return pl.BlockSpec((_BLK, w), lambda i: (i, 0))


def _full_spec(h, w):
    return pl.BlockSpec((h, w), lambda i: (0, 0))


def _tc_layer1(x, a0, a1, d0, d1, wl, wr, b):
    return pl.pallas_call(
        _tc_layer1_body,
        grid=(N // _BLK,),
        in_specs=[_row_spec(F), _row_spec(F), _row_spec(F),
                  _row_spec(1), _row_spec(1),
                  _full_spec(F, F), _full_spec(F, F), _full_spec(1, F)],
        out_specs=_row_spec(F),
        out_shape=jax.ShapeDtypeStruct((N, F), jnp.float32),
    )(x, a0, a1, d0, d1, wl, wr, b)


def _tc_layer2(x, a0, a1, d0, d1, wl, wr, b, wm, bm):
    return pl.pallas_call(
        _tc_layer2_body,
        grid=(N // _BLK,),
        in_specs=[_row_spec(F), _row_spec(F), _row_spec(F),
                  _row_spec(1), _row_spec(1),
                  _full_spec(F, F), _full_spec(F, F), _full_spec(1, F),
                  _full_spec(F, C), _full_spec(1, C)],
        out_specs=_row_spec(C),
        out_shape=jax.ShapeDtypeStruct((N, C), jnp.float32),
    )(x, a0, a1, d0, d1, wl, wr, b, wm, bm)


def kernel(x, edge_index, W1l, W1r, b1, W2l, W2r, b2, Wm, bm):
    src = edge_index[0]
    dst = edge_index[1]
    pad = EPAD - E
    src_p = jnp.concatenate([src, jnp.zeros((pad,), jnp.int32)]).reshape(NCHK, CHUNK)
    dst_p = jnp.concatenate([dst, jnp.full((pad,), N, jnp.int32)]).reshape(NCHK, CHUNK)

    agg1 = _sc_agg(x, src_p, dst_p)
    deg1 = _sc_deg(dst_p)
    a0, a1 = agg1[:N], agg1[NPAD:NPAD + N]
    d0, d1 = deg1[:N, :1], deg1[NPAD:NPAD + N, :1]

    h1 = _tc_layer1(x, a0, a1, d0, d1, W1l, W1r, b1.reshape(1, F))

    agg2 = _sc_agg(h1, src_p, dst_p)
    c0, c1 = agg2[:N], agg2[NPAD:NPAD + N]

    return _tc_layer2(h1, c0, c1, d0, d1, W2l, W2r, b2.reshape(1, F),
                      Wm, bm.reshape(1, C))


# speedup vs baseline: 7.3742x; 2.3872x over previous
import functools

import jax
import jax.numpy as jnp
from jax import lax
from jax.experimental import pallas as pl
from jax.experimental.pallas import tpu as pltpu
from jax.experimental.pallas import tpu_sc as plsc

N = 10000
E = 320000
F = 128
C = 64

NPAD = 10240
ROWS_PER_TILE = NPAD // 16
CHUNK = 128
NCHK = ((E + CHUNK - 1) // CHUNK + 255) // 256 * 256
EPAD = NCHK * CHUNK
CHUNKS_PER_TILE = NCHK // 32
CHUNKS_PER_SUBCORE = NCHK // 16
FH = F // 2
IDXB = 16
DEGW = 16


def _sc_agg_body(h0_hbm, h1_hbm, src_hbm, dst_hbm, agg_out, src_v, dst_v,
                 rows_v0, rows_v1, table_sp, acc_sp, sem0, sem1):
    cid = lax.axis_index("c")
    tid = lax.axis_index("s")

    def fill(i, _):
        for g in range(FH // 16):
            rows_v0[i, pl.ds(g * 16, 16)] = jnp.zeros((16,), jnp.float32)
        return 0
    lax.fori_loop(0, CHUNK, fill, 0)

    my0 = tid * ROWS_PER_TILE
    for k in range(ROWS_PER_TILE // CHUNK):
        pltpu.sync_copy(rows_v0, acc_sp.at[pl.ds(my0 + k * CHUNK, CHUNK)])

    @pl.when(cid == 0)
    def _():
        pltpu.sync_copy(h0_hbm.at[pl.ds(my0, ROWS_PER_TILE)],
                        table_sp.at[pl.ds(my0, ROWS_PER_TILE)])

    @pl.when(cid == 1)
    def _():
        pltpu.sync_copy(h1_hbm.at[pl.ds(my0, ROWS_PER_TILE)],
                        table_sp.at[pl.ds(my0, ROWS_PER_TILE)])
    plsc.subcore_barrier()

    def outer(g, _):
        base = tid * CHUNKS_PER_SUBCORE + g * IDXB
        pltpu.sync_copy(src_hbm.at[pl.ds(base, IDXB)], src_v)
        pltpu.sync_copy(dst_hbm.at[pl.ds(base, IDXB)], dst_v)
        pltpu.async_copy(table_sp.at[src_v.at[0]], rows_v0, sem0)

        def inner(jj, _):
            j = jj * 2
            pltpu.make_async_copy(h0_hbm.at[src_v.at[0]], rows_v0, sem0).wait()
            pltpu.async_copy(table_sp.at[src_v.at[j + 1]], rows_v1, sem1)
            pltpu.sync_copy(rows_v0, acc_sp.at[dst_v.at[j]], add=True)

            pltpu.make_async_copy(h0_hbm.at[src_v.at[0]], rows_v1, sem1).wait()

            @pl.when(jj + 1 < IDXB // 2)
            def _():
                pltpu.async_copy(table_sp.at[src_v.at[j + 2]], rows_v0, sem0)
            pltpu.sync_copy(rows_v1, acc_sp.at[dst_v.at[j + 1]], add=True)
            return 0
        lax.fori_loop(0, IDXB // 2, inner, 0)
        return 0
    lax.fori_loop(0, CHUNKS_PER_SUBCORE // IDXB, outer, 0)

    plsc.subcore_barrier()

    out0 = cid * NPAD + my0
    pltpu.sync_copy(acc_sp.at[pl.ds(my0, ROWS_PER_TILE)],
                    agg_out.at[pl.ds(out0, ROWS_PER_TILE)])


def _sc_deg_body(dst_hbm, deg_out, dst_v, ones_v, zbuf_v, deg_sp):
    cid = lax.axis_index("c")
    tid = lax.axis_index("s")
    wid = cid * 16 + tid

    def fill(i, _):
        ones_v[i, :] = jnp.ones((DEGW,), jnp.float32)
        zbuf_v[i, :] = jnp.zeros((DEGW,), jnp.float32)
        return 0
    lax.fori_loop(0, CHUNK, fill, 0)

    my0 = tid * ROWS_PER_TILE
    for k in range(ROWS_PER_TILE // CHUNK):
        pltpu.sync_copy(zbuf_v, deg_sp.at[pl.ds(my0 + k * CHUNK, CHUNK)])
    plsc.subcore_barrier()

    pltpu.sync_copy(dst_hbm.at[pl.ds(wid * CHUNKS_PER_TILE, CHUNKS_PER_TILE)], dst_v)

    def edge_body(j, _):
        pltpu.sync_copy(ones_v, deg_sp.at[dst_v.at[j]], add=True)
        return 0
    lax.fori_loop(0, CHUNKS_PER_TILE, edge_body, 0)

    plsc.subcore_barrier()

    out0 = cid * NPAD + my0
    pltpu.sync_copy(deg_sp.at[pl.ds(my0, ROWS_PER_TILE)],
                    deg_out.at[pl.ds(out0, ROWS_PER_TILE)])


_SC_PARAMS = pltpu.CompilerParams(use_tc_tiling_on_sc=False)
_SC_MESH = dict(core_axis_name="c", subcore_axis_name="s")

_sc_agg = pl.kernel(
    _sc_agg_body,
    out_type=jax.ShapeDtypeStruct((2 * NPAD, FH), jnp.float32),
    mesh=plsc.VectorSubcoreMesh(**_SC_MESH),
    scratch_types=[
        pltpu.VMEM((IDXB, CHUNK), jnp.int32),
        pltpu.VMEM((IDXB, CHUNK), jnp.int32),
        pltpu.VMEM((CHUNK, FH), jnp.float32),
        pltpu.VMEM((CHUNK, FH), jnp.float32),
        pltpu.VMEM_SHARED((NPAD, FH), jnp.float32),
        pltpu.VMEM_SHARED((NPAD, FH), jnp.float32),
        pltpu.SemaphoreType.DMA,
        pltpu.SemaphoreType.DMA,
    ],
    compiler_params=_SC_PARAMS,
)

_sc_deg = pl.kernel(
    _sc_deg_body,
    out_type=jax.ShapeDtypeStruct((2 * NPAD, DEGW), jnp.float32),
    mesh=plsc.VectorSubcoreMesh(**_SC_MESH),
    scratch_types=[
        pltpu.VMEM((CHUNKS_PER_TILE, CHUNK), jnp.int32),
        pltpu.VMEM((CHUNK, DEGW), jnp.float32),
        pltpu.VMEM((CHUNK, DEGW), jnp.float32),
        pltpu.VMEM_SHARED((NPAD, DEGW), jnp.float32),
    ],
    compiler_params=_SC_PARAMS,
)

_BLK = 2000


def _tc_layer1_body(x, a0, a1, d0, d1, wl, wr0, wr1, b, o):
    inv = 1.0 / jnp.clip(d0[...] + d1[...], 1.0, None)
    h = (jnp.dot(x[...], wl[...], preferred_element_type=jnp.float32)
         + jnp.dot(a0[...] * inv, wr0[...], preferred_element_type=jnp.float32)
         + jnp.dot(a1[...] * inv, wr1[...], preferred_element_type=jnp.float32)
         + b[...])
    o[...] = jnp.maximum(h, 0.0)


def _tc_layer2_body(x, a0, a1, d0, d1, wl, wr0, wr1, b, wm, bm, o):
    inv = 1.0 / jnp.clip(d0[...] + d1[...], 1.0, None)
    h = (jnp.dot(x[...], wl[...], preferred_element_type=jnp.float32)
         + jnp.dot(a0[...] * inv, wr0[...], preferred_element_type=jnp.float32)
         + jnp.dot(a1[...] * inv, wr1[...], preferred_element_type=jnp.float32)
         + b[...])
    h = jnp.maximum(h, 0.0)
    logits = jnp.dot(h, wm[...], preferred_element_type=jnp.float32) + bm[...]
    m = jnp.max(logits, axis=1, keepdims=True)
    s = logits - m
    lse = jnp.log(jnp.sum(jnp.exp(s), axis=1, keepdims=True))
    o[...] = s - lse


def _row_spec(w):
    return pl.BlockSpec((_BLK, w), lambda i: (i, 0))


def _full_spec(h, w):
    return pl.BlockSpec((h, w), lambda i: (0, 0))


def _tc_layer1(x, a0, a1, d0, d1, wl, wr0, wr1, b):
    return pl.pallas_call(
        _tc_layer1_body,
        grid=(N // _BLK,),
        in_specs=[_row_spec(F), _row_spec(FH), _row_spec(FH),
                  _row_spec(1), _row_spec(1),
                  _full_spec(F, F), _full_spec(FH, F), _full_spec(FH, F),
                  _full_spec(1, F)],
        out_specs=_row_spec(F),
        out_shape=jax.ShapeDtypeStruct((N, F), jnp.float32),
    )(x, a0, a1, d0, d1, wl, wr0, wr1, b)


def _tc_layer2(x, a0, a1, d0, d1, wl, wr0, wr1, b, wm, bm):
    return pl.pallas_call(
        _tc_layer2_body,
        grid=(N // _BLK,),
        in_specs=[_row_spec(F), _row_spec(FH), _row_spec(FH),
                  _row_spec(1), _row_spec(1),
                  _full_spec(F, F), _full_spec(FH, F), _full_spec(FH, F),
                  _full_spec(1, F), _full_spec(F, C), _full_spec(1, C)],
        out_specs=_row_spec(C),
        out_shape=jax.ShapeDtypeStruct((N, C), jnp.float32),
    )(x, a0, a1, d0, d1, wl, wr0, wr1, b, wm, bm)


def kernel(x, edge_index, W1l, W1r, b1, W2l, W2r, b2, Wm, bm):
    src = edge_index[0]
    dst = edge_index[1]
    pad = EPAD - E
    src_p = jnp.concatenate([src, jnp.zeros((pad,), jnp.int32)]).reshape(NCHK, CHUNK)
    dst_p = jnp.concatenate([dst, jnp.full((pad,), N, jnp.int32)]).reshape(NCHK, CHUNK)

    zrows = jnp.zeros((NPAD - N, FH), jnp.float32)
    x0 = jnp.concatenate([x[:, :FH], zrows])
    x1 = jnp.concatenate([x[:, FH:], zrows])

    agg1 = _sc_agg(x0, x1, src_p, dst_p)
    deg1 = _sc_deg(dst_p)
    a0, a1 = agg1[:N], agg1[NPAD:NPAD + N]
    d0, d1 = deg1[:N, :1], deg1[NPAD:NPAD + N, :1]

    h1 = _tc_layer1(x, a0, a1, d0, d1, W1l, W1r[:FH], W1r[FH:],
                    b1.reshape(1, F))

    h10 = jnp.concatenate([h1[:, :FH], zrows])
    h11 = jnp.concatenate([h1[:, FH:], zrows])
    agg2 = _sc_agg(h10, h11, src_p, dst_p)
    c0, c1 = agg2[:N], agg2[NPAD:NPAD + N]

    return _tc_layer2(h1, c0, c1, d0, d1, W2l, W2r[:FH], W2r[FH:],
                      b2.reshape(1, F), Wm, bm.reshape(1, C))


# speedup vs baseline: 7.9875x; 1.0832x over previous
import functools

import jax
import jax.numpy as jnp
from jax import lax
from jax.experimental import pallas as pl
from jax.experimental.pallas import tpu as pltpu
from jax.experimental.pallas import tpu_sc as plsc

N = 10000
E = 320000
F = 128
C = 64

NPAD = 10240
ROWS_PER_TILE = NPAD // 16
CHUNK = 128
NCHK = ((E + CHUNK - 1) // CHUNK + 255) // 256 * 256
EPAD = NCHK * CHUNK
CHUNKS_PER_TILE = NCHK // 32
CHUNKS_PER_SUBCORE = NCHK // 16
FH = F // 2
IDXB = 16
DEGW = 16


def _sc_agg_body(h_hbm, src_hbm, dst_hbm, agg_out, src_v, dst_v,
                 rows_v0, rows_v1, table_sp, acc_sp, sem0, sem1):
    cid = lax.axis_index("c")
    tid = lax.axis_index("s")

    def fill(i, _):
        for g in range(FH // 16):
            rows_v0[i, pl.ds(g * 16, 16)] = jnp.zeros((16,), jnp.float32)
        return 0
    lax.fori_loop(0, CHUNK, fill, 0)

    my0 = tid * ROWS_PER_TILE
    for k in range(ROWS_PER_TILE // CHUNK):
        pltpu.sync_copy(rows_v0, acc_sp.at[pl.ds(my0 + k * CHUNK, CHUNK)])

    @pl.when(cid == 0)
    def _():
        pltpu.sync_copy(h_hbm.at[pl.ds(my0, ROWS_PER_TILE), pl.ds(0, FH)],
                        table_sp.at[pl.ds(my0, ROWS_PER_TILE)])

    @pl.when(cid == 1)
    def _():
        pltpu.sync_copy(h_hbm.at[pl.ds(my0, ROWS_PER_TILE), pl.ds(FH, FH)],
                        table_sp.at[pl.ds(my0, ROWS_PER_TILE)])
    plsc.subcore_barrier()

    def outer(g, _):
        base = tid * CHUNKS_PER_SUBCORE + g * IDXB
        pltpu.sync_copy(src_hbm.at[pl.ds(base, IDXB)], src_v)
        pltpu.sync_copy(dst_hbm.at[pl.ds(base, IDXB)], dst_v)
        pltpu.async_copy(table_sp.at[src_v.at[0]], rows_v0, sem0)

        def inner(jj, _):
            j = jj * 2
            pltpu.make_async_copy(table_sp.at[src_v.at[0]], rows_v0, sem0).wait()
            pltpu.async_copy(table_sp.at[src_v.at[j + 1]], rows_v1, sem1)
            pltpu.sync_copy(rows_v0, acc_sp.at[dst_v.at[j]], add=True)

            pltpu.make_async_copy(table_sp.at[src_v.at[0]], rows_v1, sem1).wait()

            @pl.when(jj + 1 < IDXB // 2)
            def _():
                pltpu.async_copy(table_sp.at[src_v.at[j + 2]], rows_v0, sem0)
            pltpu.sync_copy(rows_v1, acc_sp.at[dst_v.at[j + 1]], add=True)
            return 0
        lax.fori_loop(0, IDXB // 2, inner, 0)
        return 0
    lax.fori_loop(0, CHUNKS_PER_SUBCORE // IDXB, outer, 0)

    plsc.subcore_barrier()

    out0 = cid * NPAD + my0
    pltpu.sync_copy(acc_sp.at[pl.ds(my0, ROWS_PER_TILE)],
                    agg_out.at[pl.ds(out0, ROWS_PER_TILE)])


def _sc_deg_body(dst_hbm, deg_out, dst_v, ones_v, zbuf_v, deg_sp):
    cid = lax.axis_index("c")
    tid = lax.axis_index("s")
    wid = cid * 16 + tid

    def fill(i, _):
        ones_v[i, :] = jnp.ones((DEGW,), jnp.float32)
        zbuf_v[i, :] = jnp.zeros((DEGW,), jnp.float32)
        return 0
    lax.fori_loop(0, CHUNK, fill, 0)

    my0 = tid * ROWS_PER_TILE
    for k in range(ROWS_PER_TILE // CHUNK):
        pltpu.sync_copy(zbuf_v, deg_sp.at[pl.ds(my0 + k * CHUNK, CHUNK)])
    plsc.subcore_barrier()

    pltpu.sync_copy(dst_hbm.at[pl.ds(wid * CHUNKS_PER_TILE, CHUNKS_PER_TILE)], dst_v)

    def edge_body(j, _):
        pltpu.sync_copy(ones_v, deg_sp.at[dst_v.at[j]], add=True)
        return 0
    lax.fori_loop(0, CHUNKS_PER_TILE, edge_body, 0)

    plsc.subcore_barrier()

    out0 = cid * NPAD + my0
    pltpu.sync_copy(deg_sp.at[pl.ds(my0, ROWS_PER_TILE)],
                    deg_out.at[pl.ds(out0, ROWS_PER_TILE)])


_SC_PARAMS = pltpu.CompilerParams(use_tc_tiling_on_sc=False)
_SC_MESH = dict(core_axis_name="c", subcore_axis_name="s")

_sc_agg = pl.kernel(
    _sc_agg_body,
    out_type=jax.ShapeDtypeStruct((2 * NPAD, FH), jnp.float32),
    mesh=plsc.VectorSubcoreMesh(**_SC_MESH),
    scratch_types=[
        pltpu.VMEM((IDXB, CHUNK), jnp.int32),
        pltpu.VMEM((IDXB, CHUNK), jnp.int32),
        pltpu.VMEM((CHUNK, FH), jnp.float32),
        pltpu.VMEM((CHUNK, FH), jnp.float32),
        pltpu.VMEM_SHARED((NPAD, FH), jnp.float32),
        pltpu.VMEM_SHARED((NPAD, FH), jnp.float32),
        pltpu.SemaphoreType.DMA,
        pltpu.SemaphoreType.DMA,
    ],
    compiler_params=_SC_PARAMS,
)

_sc_deg = pl.kernel(
    _sc_deg_body,
    out_type=jax.ShapeDtypeStruct((2 * NPAD, DEGW), jnp.float32),
    mesh=plsc.VectorSubcoreMesh(**_SC_MESH),
    scratch_types=[
        pltpu.VMEM((CHUNKS_PER_TILE, CHUNK), jnp.int32),
        pltpu.VMEM((CHUNK, DEGW), jnp.float32),
        pltpu.VMEM((CHUNK, DEGW), jnp.float32),
        pltpu.VMEM_SHARED((NPAD, DEGW), jnp.float32),
    ],
    compiler_params=_SC_PARAMS,
)

_BLK = 1280
_NB = NPAD // _BLK


def _tc_layer1_body(x, a0, a1, d0, d1, wl, wr0, wr1, b, o):
    inv = 1.0 / jnp.clip(d0[..., :1] + d1[..., :1], 1.0, None)
    h = (jnp.dot(x[...], wl[...], preferred_element_type=jnp.float32)
         + jnp.dot(a0[...] * inv, wr0[...], preferred_element_type=jnp.float32)
         + jnp.dot(a1[...] * inv, wr1[...], preferred_element_type=jnp.float32)
         + b[...])
    o[...] = jnp.maximum(h, 0.0)


def _tc_layer2_body(x, a0, a1, d0, d1, wl, wr0, wr1, b, wm, bm, o):
    inv = 1.0 / jnp.clip(d0[..., :1] + d1[..., :1], 1.0, None)
    h = (jnp.dot(x[...], wl[...], preferred_element_type=jnp.float32)
         + jnp.dot(a0[...] * inv, wr0[...], preferred_element_type=jnp.float32)
         + jnp.dot(a1[...] * inv, wr1[...], preferred_element_type=jnp.float32)
         + b[...])
    h = jnp.maximum(h, 0.0)
    logits = jnp.dot(h, wm[...], preferred_element_type=jnp.float32) + bm[...]
    m = jnp.max(logits, axis=1, keepdims=True)
    s = logits - m
    lse = jnp.log(jnp.sum(jnp.exp(s), axis=1, keepdims=True))
    o[...] = s - lse


def _row_spec(w):
    return pl.BlockSpec((_BLK, w), lambda i: (i, 0))


def _hi_spec(w):
    return pl.BlockSpec((_BLK, w), lambda i: (i + _NB, 0))


def _full_spec(h, w):
    return pl.BlockSpec((h, w), lambda i: (0, 0))


def _tc_layer1(x, agg, deg, wl, wr0, wr1, b):
    return pl.pallas_call(
        _tc_layer1_body,
        grid=(_NB,),
        in_specs=[_row_spec(F), _row_spec(FH), _hi_spec(FH),
                  _row_spec(DEGW), _hi_spec(DEGW),
                  _full_spec(F, F), _full_spec(FH, F), _full_spec(FH, F),
                  _full_spec(1, F)],
        out_specs=_row_spec(F),
        out_shape=jax.ShapeDtypeStruct((NPAD, F), jnp.float32),
    )(x, agg, agg, deg, deg, wl, wr0, wr1, b)


def _tc_layer2(x, agg, deg, wl, wr0, wr1, b, wm, bm):
    return pl.pallas_call(
        _tc_layer2_body,
        grid=(_NB,),
        in_specs=[_row_spec(F), _row_spec(FH), _hi_spec(FH),
                  _row_spec(DEGW), _hi_spec(DEGW),
                  _full_spec(F, F), _full_spec(FH, F), _full_spec(FH, F),
                  _full_spec(1, F), _full_spec(F, C), _full_spec(1, C)],
        out_specs=_row_spec(C),
        out_shape=jax.ShapeDtypeStruct((NPAD, C), jnp.float32),
    )(x, agg, agg, deg, deg, wl, wr0, wr1, b, wm, bm)


def kernel(x, edge_index, W1l, W1r, b1, W2l, W2r, b2, Wm, bm):
    src = edge_index[0]
    dst = edge_index[1]
    pad = EPAD - E
    src_p = jnp.concatenate([src, jnp.zeros((pad,), jnp.int32)]).reshape(NCHK, CHUNK)
    dst_p = jnp.concatenate([dst, jnp.full((pad,), N, jnp.int32)]).reshape(NCHK, CHUNK)

    xp = jnp.concatenate([x, jnp.zeros((NPAD - N, F), jnp.float32)])

    agg1 = _sc_agg(xp, src_p, dst_p)
    deg = _sc_deg(dst_p)

    h1 = _tc_layer1(xp, agg1, deg, W1l, W1r[:FH], W1r[FH:], b1.reshape(1, F))

    agg2 = _sc_agg(h1, src_p, dst_p)

    out = _tc_layer2(h1, agg2, deg, W2l, W2r[:FH], W2r[FH:],
                     b2.reshape(1, F), Wm, bm.reshape(1, C))
    return out[:N]
